# baseline (speedup 1.0000x reference)
import numpy as np
import jax
import jax.numpy as jnp
from jax.experimental import pallas as pl
from jax.experimental.pallas import tpu as pltpu

_B, _N, _DIM, _H = 16, 197, 768, 12
_d = _DIM // _H
_M = 25
_KVT = 25
_GRID_HW = 14
_POOL = 5
_SCALE = _d ** -0.5
_HIGH = jax.lax.Precision.HIGHEST
_BF = jnp.bfloat16
_NEG = float('-inf')


def _dot(a, b, dims):
    return jax.lax.dot_general(a, b, (dims, ((), ())),
                               preferred_element_type=jnp.float32)


def _pool_mat() -> np.ndarray:
    P = np.zeros((_POOL, _GRID_HW), np.float32)
    for i in range(_POOL):
        s = (i * _GRID_HW) // _POOL
        e = -((-(i + 1) * _GRID_HW) // _POOL)
        P[i, s:e] = 1.0 / (e - s)
    A = np.einsum('ph,qw->pqhw', P, P).reshape(_M, _GRID_HW * _GRID_HW)
    return np.concatenate([A, np.zeros((_M, 1), np.float32)], axis=1)


def _fused_kernel(x_ref, w_ref, a_ref, lt_ref, ones_ref, wp_ref, b_ref, o_ref):
    x = x_ref[0]
    q_full = _dot(x, w_ref[0], ((1,), (1,)))
    k_full = _dot(x, w_ref[1], ((1,), (1,)))
    v_full = _dot(x, w_ref[2], ((1,), (1,)))
    qb_full = q_full.astype(_BF)
    kb_full = k_full.astype(_BF)
    vb_full = v_full.astype(_BF)

    r_cat = jax.lax.dot_general(a_ref[...], q_full, (((1,), (0,)), ((), ())),
                                precision=_HIGH,
                                preferred_element_type=jnp.float32)
    rb_cat = r_cat.astype(_BF)

    qb = [qb_full[:, h * _d:(h + 1) * _d] for h in range(_H)]
    kb = [kb_full[:, h * _d:(h + 1) * _d] for h in range(_H)]
    vb = [vb_full[:, h * _d:(h + 1) * _d] for h in range(_H)]
    rb = [rb_cat[:, h * _d:(h + 1) * _d] for h in range(_H)]

    rk = [_dot(rb[h], kb[h], ((1,), (1,))) for h in range(_H)]

    def body(_, carry):
        curs, cnts, thrs = carry
        ncur, ncnt, nthr = [], [], []
        for cur, cnt, thr in zip(curs, cnts, thrs):
            rowmax = jnp.max(cur, axis=1, keepdims=True)
            tie = (cur == rowmax).astype(_BF)
            cnt2 = cnt + _dot(tie, ones_ref[...], ((1,), (0,)))
            nthr.append(jnp.where((cnt < _KVT) & (cnt2 >= _KVT),
                                  rowmax, thr))
            ncur.append(jnp.where(tie > 0.5, _NEG, cur))
            ncnt.append(cnt2)
        return tuple(ncur), tuple(ncnt), tuple(nthr)

    zero1 = jnp.zeros((_M, 1), jnp.float32)
    init = (tuple(rk), (zero1,) * _H, (jnp.full((_M, 1), jnp.inf),) * _H)
    _, _, thrs = jax.lax.fori_loop(0, _KVT, body, init)

    keymask = []
    for h in range(_H):
        gt = (rk[h] > thrs[h]).astype(_BF)
        tie = (rk[h] == thrs[h]).astype(_BF)
        c = _dot(gt, ones_ref[...], ((1,), (0,)))
        prefix = _dot(tie, lt_ref[...], ((1,), (0,)))
        ok = jnp.logical_and(tie > 0.5, prefix <= (_KVT - c))
        keymask.append(jnp.maximum(gt, ok.astype(_BF)))

    iota_m = jax.lax.broadcasted_iota(jnp.int32, (_M, _N), 0)
    out_acc = b_ref[...]
    for h in range(_H):
        s_rk = rk[h] * _SCALE
        s_rk = s_rk - jnp.max(s_rk, axis=1, keepdims=True)
        e_rk = jnp.exp(s_rk)
        agent_p = e_rk / jnp.sum(e_rk, axis=1, keepdims=True)
        agent_value = _dot(agent_p.astype(_BF), vb[h], ((1,), (0,)))

        gate = _dot(rb[h], qb[h], ((1,), (1,)))
        colmax = jnp.max(gate, axis=0, keepdims=True)
        eidx = jnp.min(jnp.where(gate == colmax, iota_m, _M), axis=0,
                       keepdims=True)
        onehot_e = (iota_m == eidx).astype(_BF)
        qmask = _dot(onehot_e, keymask[h], ((0,), (0,)))
        sel = qmask > 0.5

        al = _SCALE * _dot(qb[h], rb[h], ((1,), (1,)))
        s = _SCALE * _dot(qb[h], kb[h], ((1,), (1,)))
        s_m = jnp.where(sel, s, _NEG)
        mx = jnp.maximum(jnp.max(al, axis=1, keepdims=True),
                         jnp.max(s_m, axis=1, keepdims=True))
        e_a = jnp.exp(al - mx)
        e_s = jnp.exp(s_m - mx)
        denom = (jnp.sum(e_a, axis=1, keepdims=True)
                 + jnp.sum(e_s, axis=1, keepdims=True))
        out64 = (_dot((e_a / denom).astype(_BF), agent_value.astype(_BF),
                      ((1,), (0,)))
                 + _dot((e_s / denom).astype(_BF), vb[h], ((1,), (0,))))

        out_acc = out_acc + _dot(out64.astype(_BF),
                                 wp_ref[h * _d:(h + 1) * _d, :],
                                 ((1,), (0,)))
    o_ref[0] = out_acc


def kernel(x, W_qkv, W_proj, b_proj):
    A = jnp.asarray(_pool_mat())
    LT = jnp.asarray(np.triu(np.ones((_N, _N), np.float32))).astype(_BF)
    ONES = jnp.ones((_N, 1), _BF)
    W3 = W_qkv.reshape(3, _DIM, _DIM).astype(_BF)
    Wp = W_proj.T.astype(_BF)

    out = pl.pallas_call(
        _fused_kernel,
        grid=(_B,),
        in_specs=[
            pl.BlockSpec((1, _N, _DIM), lambda b: (b, 0, 0)),
            pl.BlockSpec((3, _DIM, _DIM), lambda b: (0, 0, 0)),
            pl.BlockSpec((_M, _N), lambda b: (0, 0)),
            pl.BlockSpec((_N, _N), lambda b: (0, 0)),
            pl.BlockSpec((_N, 1), lambda b: (0, 0)),
            pl.BlockSpec((_DIM, _DIM), lambda b: (0, 0)),
            pl.BlockSpec((1, _DIM), lambda b: (0, 0)),
        ],
        out_specs=pl.BlockSpec((1, _N, _DIM), lambda b: (b, 0, 0)),
        out_shape=jax.ShapeDtypeStruct((_B, _N, _DIM), jnp.float32),
        compiler_params=pltpu.CompilerParams(
            dimension_semantics=("parallel",)),
    )(x.astype(_BF), W3, A, LT, ONES, Wp, b_proj.reshape(1, _DIM))

    return out


# speedup vs baseline: 1.2067x; 1.2067x over previous
import numpy as np
import jax
import jax.numpy as jnp
from jax.experimental import pallas as pl
from jax.experimental.pallas import tpu as pltpu

_B, _N, _DIM, _H = 16, 197, 768, 12
_d = _DIM // _H
_M = 25
_KVT = 25
_GRID_HW = 14
_POOL = 5
_SCALE = _d ** -0.5
_HIGH = jax.lax.Precision.HIGHEST
_BF = jnp.bfloat16
_NEG = float('-inf')


def _dot(a, b, dims):
    return jax.lax.dot_general(a, b, (dims, ((), ())),
                               preferred_element_type=jnp.float32)


def _pool_mat() -> np.ndarray:
    P = np.zeros((_POOL, _GRID_HW), np.float32)
    for i in range(_POOL):
        s = (i * _GRID_HW) // _POOL
        e = -((-(i + 1) * _GRID_HW) // _POOL)
        P[i, s:e] = 1.0 / (e - s)
    A = np.einsum('ph,qw->pqhw', P, P).reshape(_M, _GRID_HW * _GRID_HW)
    return np.concatenate([A, np.zeros((_M, 1), np.float32)], axis=1)


def _fused_kernel(x_ref, w_ref, a_ref, wp_ref, b_ref, o_ref):
    x = x_ref[0]
    q_full = _dot(x, w_ref[0], ((1,), (1,)))
    k_full = _dot(x, w_ref[1], ((1,), (1,)))
    v_full = _dot(x, w_ref[2], ((1,), (1,)))
    qb_full = q_full.astype(_BF)
    kb_full = k_full.astype(_BF)
    vb_full = v_full.astype(_BF)

    r_cat = jax.lax.dot_general(a_ref[...], q_full, (((1,), (0,)), ((), ())),
                                precision=_HIGH,
                                preferred_element_type=jnp.float32)
    rb_cat = r_cat.astype(_BF)

    qb = [qb_full[:, h * _d:(h + 1) * _d] for h in range(_H)]
    kb = [kb_full[:, h * _d:(h + 1) * _d] for h in range(_H)]
    vb = [vb_full[:, h * _d:(h + 1) * _d] for h in range(_H)]
    rb = [rb_cat[:, h * _d:(h + 1) * _d] for h in range(_H)]

    rk = [_dot(rb[h], kb[h], ((1,), (1,))) for h in range(_H)]

    iota_n3 = jax.lax.broadcasted_iota(jnp.int32, (_H, _M, _N), 2)

    def body(_, cur):
        rowmax = jnp.max(cur, axis=2, keepdims=True)
        idx = jnp.min(jnp.where(cur == rowmax, iota_n3, _N), axis=2,
                      keepdims=True)
        return jnp.where(iota_n3 == idx, _NEG, cur)

    curs = jax.lax.fori_loop(0, _KVT, body, jnp.stack(rk, axis=0))
    keymask = [(curs[h] == _NEG).astype(_BF) for h in range(_H)]

    iota_m = jax.lax.broadcasted_iota(jnp.int32, (_M, _N), 0)
    out_acc = b_ref[...]
    for h in range(_H):
        s_rk = rk[h] * _SCALE
        s_rk = s_rk - jnp.max(s_rk, axis=1, keepdims=True)
        e_rk = jnp.exp(s_rk)
        agent_p = e_rk / jnp.sum(e_rk, axis=1, keepdims=True)
        agent_value = _dot(agent_p.astype(_BF), vb[h], ((1,), (0,)))

        gate = _dot(rb[h], qb[h], ((1,), (1,)))
        colmax = jnp.max(gate, axis=0, keepdims=True)
        eidx = jnp.min(jnp.where(gate == colmax, iota_m, _M), axis=0,
                       keepdims=True)
        onehot_e = (iota_m == eidx).astype(_BF)
        qmask = _dot(onehot_e, keymask[h], ((0,), (0,)))
        sel = qmask > 0.5

        al = _SCALE * _dot(qb[h], rb[h], ((1,), (1,)))
        s = _SCALE * _dot(qb[h], kb[h], ((1,), (1,)))
        s_m = jnp.where(sel, s, _NEG)
        mx = jnp.maximum(jnp.max(al, axis=1, keepdims=True),
                         jnp.max(s_m, axis=1, keepdims=True))
        e_a = jnp.exp(al - mx)
        e_s = jnp.exp(s_m - mx)
        denom = (jnp.sum(e_a, axis=1, keepdims=True)
                 + jnp.sum(e_s, axis=1, keepdims=True))
        out64 = (_dot((e_a / denom).astype(_BF), agent_value.astype(_BF),
                      ((1,), (0,)))
                 + _dot((e_s / denom).astype(_BF), vb[h], ((1,), (0,))))

        out_acc = out_acc + _dot(out64.astype(_BF),
                                 wp_ref[h * _d:(h + 1) * _d, :],
                                 ((1,), (0,)))
    o_ref[0] = out_acc


def kernel(x, W_qkv, W_proj, b_proj):
    A = jnp.asarray(_pool_mat())
    W3 = W_qkv.reshape(3, _DIM, _DIM).astype(_BF)
    Wp = W_proj.T.astype(_BF)

    out = pl.pallas_call(
        _fused_kernel,
        grid=(_B,),
        in_specs=[
            pl.BlockSpec((1, _N, _DIM), lambda b: (b, 0, 0)),
            pl.BlockSpec((3, _DIM, _DIM), lambda b: (0, 0, 0)),
            pl.BlockSpec((_M, _N), lambda b: (0, 0)),
            pl.BlockSpec((_DIM, _DIM), lambda b: (0, 0)),
            pl.BlockSpec((1, _DIM), lambda b: (0, 0)),
        ],
        out_specs=pl.BlockSpec((1, _N, _DIM), lambda b: (b, 0, 0)),
        out_shape=jax.ShapeDtypeStruct((_B, _N, _DIM), jnp.float32),
        compiler_params=pltpu.CompilerParams(
            dimension_semantics=("parallel",)),
    )(x.astype(_BF), W3, A, Wp, b_proj.reshape(1, _DIM))

    return out
